# baseline (speedup 1.0000x reference)
import functools

import jax
import jax.numpy as jnp
from jax import lax
from jax.experimental import pallas as pl
from jax.experimental.pallas import tpu as pltpu
from jax.experimental.pallas import tpu_sc as plsc

NC = 2
NS = 16
NW = NC * NS
CHUNK = 128
NBUF = 5
LAG = 2
ROW = 128


def _make_lookup(B, H, D):
    total = B * H
    K = total // CHUNK
    kpt = K // NW
    n_groups = kpt // NBUF
    assert total % (NW * CHUNK) == 0 and kpt % NBUF == 0 and n_groups >= 3
    mesh = plsc.VectorSubcoreMesh(
        core_axis_name="c", subcore_axis_name="s",
        num_cores=NC, num_subcores=NS,
    )

    @functools.partial(
        pl.kernel,
        mesh=mesh,
        out_type=jax.ShapeDtypeStruct((K, CHUNK, D), jnp.float32),
        scratch_types=[
            pltpu.VMEM((kpt, CHUNK), jnp.int32),
            pltpu.VMEM((NBUF, CHUNK, ROW), jnp.float32),
            pltpu.SemaphoreType.DMA((NBUF,)),
            pltpu.SemaphoreType.DMA((NBUF,)),
        ],
        compiler_params=pltpu.CompilerParams(
            use_tc_tiling_on_sc=False, needs_layout_passes=False
        ),
    )
    def run(idx_hbm, table_hbm, out_hbm, idx_v, rows_v, gsem, ssem):
        wid = lax.axis_index("s") * NC + lax.axis_index("c")
        k0 = wid * kpt
        pltpu.sync_copy(idx_hbm.at[pl.ds(k0, kpt)], idx_v)

        def gather_start(k, b):
            pltpu.async_copy(
                table_hbm.at[idx_v.at[k]], rows_v.at[b], gsem.at[b]
            )

        def gather_wait(b):
            pltpu.make_async_copy(
                table_hbm.at[pl.ds(0, CHUNK)], rows_v.at[b], gsem.at[b]
            ).wait()

        def store_start(k, b):
            pltpu.async_copy(
                rows_v.at[b, :, pl.ds(0, D)], out_hbm.at[k0 + k], ssem.at[b]
            )

        def store_wait(b):
            pltpu.make_async_copy(
                table_hbm.at[pl.ds(0, CHUNK // 2)],
                rows_v.at[b, pl.ds(0, CHUNK // 2)],
                ssem.at[b],
            ).wait()

        for b in range(NBUF):
            gather_start(b, b)

        for k in range(NBUF):
            gather_wait(k)
            store_start(k, k)
            if k >= LAG:
                b2 = k - LAG
                store_wait(b2)
                gather_start(k - LAG + NBUF, b2)

        def group(g, _):
            for b in range(NBUF):
                k = g * NBUF + b
                gather_wait(b)
                store_start(k, b)
                b2 = (b - LAG) % NBUF
                store_wait(b2)
                gather_start(k - LAG + NBUF, b2)
            return 0

        lax.fori_loop(1, n_groups - 1, group, 0)

        for b in range(NBUF):
            k = (n_groups - 1) * NBUF + b
            gather_wait(b)
            store_start(k, b)
            if b < LAG:
                b2 = (b - LAG) % NBUF
                store_wait(b2)
                gather_start(k - LAG + NBUF, b2)
        for b in range(NBUF):
            store_wait(b)

    return run


def kernel(input_tokens, table):
    B, H = input_tokens.shape
    V, D = table.shape
    idx = input_tokens.astype(jnp.int32).reshape(-1, CHUNK)
    tbl = jnp.pad(table, ((0, 0), (0, 128 - D)))
    x = _make_lookup(B, H, D)(idx, tbl)
    return x.reshape(B, H, D)
